# baseline (speedup 1.0000x reference)
import functools

import jax
import jax.numpy as jnp
from jax import lax
from jax.experimental import pallas as pl
from jax.experimental.pallas import tpu as pltpu
from jax.experimental.pallas import tpu_sc as plsc

EMBED = 64
NUM_CORES = 2
NUM_SUBCORES = 16
NUM_WORKERS = NUM_CORES * NUM_SUBCORES
CHUNK = 320
LANES = 16


def _make_flatten(batch: int, hist: int):
    rows_per_w = batch // NUM_WORKERS
    b_per_w = rows_per_w * hist
    mesh = plsc.VectorSubcoreMesh(core_axis_name="c", subcore_axis_name="s")

    @functools.partial(
        pl.kernel,
        mesh=mesh,
        out_type=jax.ShapeDtypeStruct((batch * hist,), jnp.int32),
        scratch_types=[
            pltpu.VMEM((hist, rows_per_w), jnp.int32),
            pltpu.VMEM((b_per_w,), jnp.int32),
        ],
        compiler_params=pltpu.CompilerParams(
            use_tc_tiling_on_sc=True, needs_layout_passes=False),
    )
    def flatten(idxt_hbm, out_hbm, idx2d_v, idxf_v):
        wid = lax.axis_index("s") * NUM_CORES + lax.axis_index("c")
        rbase = wid * rows_per_w
        pltpu.sync_copy(idxt_hbm.at[:, pl.ds(rbase, rows_per_w)], idx2d_v)

        def tbody(g, hr):
            h, r = hr
            idxf_v[pl.ds(g * LANES, LANES)] = plsc.load_gather(idx2d_v, [h, r])
            h2 = h + LANES
            m = h2 >= hist
            h2 = jnp.where(m, h2 - hist, h2)
            r2 = jnp.where(m, r + 1, r)
            return (h2, r2)

        h0 = lax.iota(jnp.int32, LANES)
        r0 = jnp.zeros((LANES,), jnp.int32)
        lax.fori_loop(0, b_per_w // LANES, tbody, (h0, r0))
        pltpu.sync_copy(idxf_v, out_hbm.at[pl.ds(wid * b_per_w, b_per_w)])

    return flatten


TPAD = 1000448


def _make_table_prep(vocab_rows: int):
    del vocab_rows
    grid = TPAD // 512

    return pl.pallas_call(
        _table_prep_body,
        grid=(grid,),
        in_specs=[pl.BlockSpec((EMBED, 512), lambda j: (0, j))],
        out_specs=pl.BlockSpec((512, 128), lambda j: (j, 0)),
        out_shape=jax.ShapeDtypeStruct((TPAD, 128), jnp.float32),
    )


def _table_prep_body(tt_ref, out_ref):
    out_ref[:, :EMBED] = tt_ref[...].T
    out_ref[:, EMBED:] = jnp.zeros((512, EMBED), jnp.float32)


def _make_lookup(batch: int, hist: int):
    total = batch * hist
    b_per_w = total // NUM_WORKERS
    n_pairs = b_per_w // (2 * CHUNK)
    mesh = plsc.VectorSubcoreMesh(core_axis_name="c", subcore_axis_name="s")

    @functools.partial(
        pl.kernel,
        mesh=mesh,
        out_type=jax.ShapeDtypeStruct((batch * hist, EMBED), jnp.float32),
        scratch_types=[
            pltpu.VMEM((b_per_w,), jnp.int32),
            pltpu.VMEM((2, CHUNK, 128), jnp.float32),
            pltpu.SemaphoreType.DMA,
            pltpu.SemaphoreType.DMA,
            pltpu.SemaphoreType.DMA,
            pltpu.SemaphoreType.DMA,
        ],
        compiler_params=pltpu.CompilerParams(use_tc_tiling_on_sc=False),
    )
    def lookup(table_hbm, idx_hbm, out_hbm, idx_v, rows_v, gsem_a, gsem_b,
               osem_a, osem_b):
        wid = lax.axis_index("s") * NUM_CORES + lax.axis_index("c")
        base = wid * b_per_w
        pltpu.sync_copy(idx_hbm.at[pl.ds(base, b_per_w)], idx_v)

        buf_a = rows_v.at[0]
        buf_b = rows_v.at[1]

        def gat(chunk, buf, sem):
            return pltpu.make_async_copy(
                table_hbm.at[idx_v.at[pl.ds(chunk * CHUNK, CHUNK)]], buf, sem)

        def out(chunk, buf, sem):
            return pltpu.make_async_copy(
                buf.at[:, 0:EMBED],
                out_hbm.at[pl.ds(base + chunk * CHUNK, CHUNK)], sem)

        gat(0, buf_a, gsem_a).start()

        def body(p, carry):
            c0 = 2 * p
            c1 = c0 + 1

            @pl.when(p > 0)
            def _():
                out(c1 - 2, buf_b, osem_b).wait()

            gat(c1, buf_b, gsem_b).start()
            gat(c0, buf_a, gsem_a).wait()
            out(c0, buf_a, osem_a).start()
            gat(c1, buf_b, gsem_b).wait()
            out(c1, buf_b, osem_b).start()

            @pl.when(p < n_pairs - 1)
            def _():
                out(c0, buf_a, osem_a).wait()
                gat(c0 + 2, buf_a, gsem_a).start()

            return carry

        lax.fori_loop(0, n_pairs, body, 0)
        out(2 * n_pairs - 2, buf_a, osem_a).wait()
        out(2 * n_pairs - 1, buf_b, osem_b).wait()

    return lookup


def kernel(entity_table, type_index):
    batch, hist = type_index.shape
    idx_t = type_index.astype(jnp.int32).T
    idx_flat = _make_flatten(batch, hist)(idx_t)
    table128 = _make_table_prep(entity_table.shape[0])(entity_table.T)
    out = _make_lookup(batch, hist)(table128, idx_flat)
    return out.reshape(batch, hist, EMBED)


# speedup vs baseline: 1.5977x; 1.5977x over previous
import functools

import jax
import jax.numpy as jnp
from jax import lax
from jax.experimental import pallas as pl
from jax.experimental.pallas import tpu as pltpu
from jax.experimental.pallas import tpu_sc as plsc

EMBED = 64
NUM_CORES = 2
NUM_SUBCORES = 16
NUM_WORKERS = NUM_CORES * NUM_SUBCORES
CHUNK = 640
LANES = 16


def _make_flatten(batch: int, hist: int):
    rows_per_w = batch // NUM_WORKERS
    b_per_w = rows_per_w * hist
    mesh = plsc.VectorSubcoreMesh(core_axis_name="c", subcore_axis_name="s")

    @functools.partial(
        pl.kernel,
        mesh=mesh,
        out_type=jax.ShapeDtypeStruct((batch * hist,), jnp.int32),
        scratch_types=[
            pltpu.VMEM((hist, rows_per_w), jnp.int32),
            pltpu.VMEM((b_per_w,), jnp.int32),
        ],
        compiler_params=pltpu.CompilerParams(
            use_tc_tiling_on_sc=True, needs_layout_passes=False),
    )
    def flatten(idxt_hbm, out_hbm, idx2d_v, idxf_v):
        wid = lax.axis_index("s") * NUM_CORES + lax.axis_index("c")
        rbase = wid * rows_per_w
        pltpu.sync_copy(idxt_hbm.at[:, pl.ds(rbase, rows_per_w)], idx2d_v)

        def tbody(g, hr):
            h, r = hr
            idxf_v[pl.ds(g * LANES, LANES)] = plsc.load_gather(idx2d_v, [h, r])
            h2 = h + LANES
            m = h2 >= hist
            h2 = jnp.where(m, h2 - hist, h2)
            r2 = jnp.where(m, r + 1, r)
            return (h2, r2)

        h0 = lax.iota(jnp.int32, LANES)
        r0 = jnp.zeros((LANES,), jnp.int32)
        lax.fori_loop(0, b_per_w // LANES, tbody, (h0, r0))
        pltpu.sync_copy(idxf_v, out_hbm.at[pl.ds(wid * b_per_w, b_per_w)])

    return flatten


def _make_lookup(batch: int, hist: int):
    total = batch * hist
    b_per_w = total // NUM_WORKERS
    n_pairs = b_per_w // (2 * CHUNK)
    mesh = plsc.VectorSubcoreMesh(core_axis_name="c", subcore_axis_name="s")

    @functools.partial(
        pl.kernel,
        mesh=mesh,
        out_type=jax.ShapeDtypeStruct((batch * hist, EMBED), jnp.float32),
        scratch_types=[
            pltpu.VMEM((b_per_w,), jnp.int32),
            pltpu.VMEM((2, CHUNK, EMBED), jnp.float32),
            pltpu.SemaphoreType.DMA,
            pltpu.SemaphoreType.DMA,
            pltpu.SemaphoreType.DMA,
            pltpu.SemaphoreType.DMA,
        ],
        compiler_params=pltpu.CompilerParams(use_tc_tiling_on_sc=False),
    )
    def lookup(table_hbm, idx_hbm, out_hbm, idx_v, rows_v, gsem_a, gsem_b,
               osem_a, osem_b):
        wid = lax.axis_index("s") * NUM_CORES + lax.axis_index("c")
        base = wid * b_per_w
        pltpu.sync_copy(idx_hbm.at[pl.ds(base, b_per_w)], idx_v)

        buf_a = rows_v.at[0]
        buf_b = rows_v.at[1]

        def gat(chunk, buf, sem):
            return pltpu.make_async_copy(
                table_hbm.at[idx_v.at[pl.ds(chunk * CHUNK, CHUNK)]], buf, sem)

        def out(chunk, buf, sem):
            return pltpu.make_async_copy(
                buf, out_hbm.at[pl.ds(base + chunk * CHUNK, CHUNK)], sem)

        gat(0, buf_a, gsem_a).start()

        def body(p, carry):
            c0 = 2 * p
            c1 = c0 + 1

            @pl.when(p > 0)
            def _():
                out(c1 - 2, buf_b, osem_b).wait()

            gat(c1, buf_b, gsem_b).start()
            gat(c0, buf_a, gsem_a).wait()
            out(c0, buf_a, osem_a).start()
            gat(c1, buf_b, gsem_b).wait()
            out(c1, buf_b, osem_b).start()

            @pl.when(p < n_pairs - 1)
            def _():
                out(c0, buf_a, osem_a).wait()
                gat(c0 + 2, buf_a, gsem_a).start()

            return carry

        lax.fori_loop(0, n_pairs, body, 0)
        out(2 * n_pairs - 2, buf_a, osem_a).wait()
        out(2 * n_pairs - 1, buf_b, osem_b).wait()

    return lookup


def kernel(entity_table, type_index):
    batch, hist = type_index.shape
    idx_t = type_index.astype(jnp.int32).T
    idx_flat = _make_flatten(batch, hist)(idx_t)
    out = _make_lookup(batch, hist)(entity_table, idx_flat)
    return out.reshape(batch, hist, EMBED)


# speedup vs baseline: 1.6846x; 1.0544x over previous
import functools

import jax
import jax.numpy as jnp
from jax import lax
from jax.experimental import pallas as pl
from jax.experimental.pallas import tpu as pltpu
from jax.experimental.pallas import tpu_sc as plsc

EMBED = 64
NUM_CORES = 2
NUM_SUBCORES = 16
NUM_WORKERS = NUM_CORES * NUM_SUBCORES
CHUNK = 512
LANES = 16


def _make_flatten(batch: int, hist: int):
    rows_per_w = batch // NUM_WORKERS
    b_per_w = rows_per_w * hist
    mesh = plsc.VectorSubcoreMesh(core_axis_name="c", subcore_axis_name="s")

    @functools.partial(
        pl.kernel,
        mesh=mesh,
        out_type=jax.ShapeDtypeStruct((batch * hist,), jnp.int32),
        scratch_types=[
            pltpu.VMEM((hist, rows_per_w), jnp.int32),
            pltpu.VMEM((b_per_w,), jnp.int32),
        ],
        compiler_params=pltpu.CompilerParams(
            use_tc_tiling_on_sc=True, needs_layout_passes=False),
    )
    def flatten(idxt_hbm, out_hbm, idx2d_v, idxf_v):
        wid = lax.axis_index("s") * NUM_CORES + lax.axis_index("c")
        rbase = wid * rows_per_w
        pltpu.sync_copy(idxt_hbm.at[:, pl.ds(rbase, rows_per_w)], idx2d_v)

        def tbody(g, hr):
            h, r = hr
            idxf_v[pl.ds(g * LANES, LANES)] = plsc.load_gather(idx2d_v, [h, r]) * 2
            h2 = h + LANES
            m = h2 >= hist
            h2 = jnp.where(m, h2 - hist, h2)
            r2 = jnp.where(m, r + 1, r)
            return (h2, r2)

        h0 = lax.iota(jnp.int32, LANES)
        r0 = jnp.zeros((LANES,), jnp.int32)
        lax.fori_loop(0, b_per_w // LANES, tbody, (h0, r0))
        pltpu.sync_copy(idxf_v, out_hbm.at[pl.ds(wid * b_per_w, b_per_w)])

    return flatten


def _make_lookup(batch: int, hist: int):
    total = batch * hist
    b_per_w = total // NUM_WORKERS
    n_pairs = b_per_w // (2 * CHUNK)
    mesh = plsc.VectorSubcoreMesh(core_axis_name="c", subcore_axis_name="s")

    @functools.partial(
        pl.kernel,
        mesh=mesh,
        out_type=jax.ShapeDtypeStruct((batch * hist, EMBED), jnp.float32),
        scratch_types=[
            pltpu.VMEM((b_per_w,), jnp.int32),
            pltpu.VMEM((2, CHUNK, EMBED), jnp.float32),
            pltpu.SemaphoreType.DMA,
            pltpu.SemaphoreType.DMA,
            pltpu.SemaphoreType.DMA,
            pltpu.SemaphoreType.DMA,
        ],
        compiler_params=pltpu.CompilerParams(use_tc_tiling_on_sc=False),
    )
    def lookup(table_hbm, idx_hbm, out_hbm, idx_v, rows_v, gsem_a, gsem_b,
               osem_a, osem_b):
        wid = lax.axis_index("s") * NUM_CORES + lax.axis_index("c")
        base = wid * b_per_w
        pltpu.sync_copy(idx_hbm.at[pl.ds(base, b_per_w)], idx_v)

        buf_a = rows_v.at[0]
        buf_b = rows_v.at[1]

        def gat(chunk, buf, sem):
            return pltpu.make_async_copy(
                table_hbm.at[idx_v.at[pl.ds(chunk * CHUNK, CHUNK)]], buf, sem)

        def out(chunk, buf, sem):
            return pltpu.make_async_copy(
                buf, out_hbm.at[pl.ds(base + chunk * CHUNK, CHUNK)], sem)

        gat(0, buf_a, gsem_a).start()

        def body(p, carry):
            c0 = 2 * p
            c1 = c0 + 1

            @pl.when(p > 0)
            def _():
                out(c1 - 2, buf_b, osem_b).wait()

            gat(c1, buf_b, gsem_b).start()
            gat(c0, buf_a, gsem_a).wait()
            out(c0, buf_a, osem_a).start()
            gat(c1, buf_b, gsem_b).wait()
            out(c1, buf_b, osem_b).start()

            @pl.when(p < n_pairs - 1)
            def _():
                out(c0, buf_a, osem_a).wait()
                gat(c0 + 2, buf_a, gsem_a).start()

            return carry

        lax.fori_loop(0, n_pairs, body, 0)
        out(2 * n_pairs - 2, buf_a, osem_a).wait()
        out(2 * n_pairs - 1, buf_b, osem_b).wait()

    return lookup


def kernel(entity_table, type_index):
    batch, hist = type_index.shape
    idx_t = type_index.astype(jnp.int32).T
    idx_flat = _make_flatten(batch, hist)(idx_t)
    vocab = entity_table.shape[0]
    vpad = (vocab + 7) // 8 * 8
    tpad = jnp.pad(entity_table, ((0, vpad - vocab), (0, 128 - EMBED)))
    tview = tpad.reshape(vpad * 2, EMBED)
    out = _make_lookup(batch, hist)(tview, idx_flat)
    return out.reshape(batch, hist, EMBED)
